# baseline (speedup 1.0000x reference)
import functools

import jax
import jax.numpy as jnp
from jax import lax
from jax.experimental import pallas as pl
from jax.experimental.pallas import tpu as pltpu
from jax.experimental.pallas import tpu_sc as plsc

VOCAB = 120
EMBED_DIM = 200
LANE = 128
TAIL = EMBED_DIM - LANE
NBUF = 4
TC_ROWS = 128


def _sc_head_inplace(out_ref, idx3, table_a):
    info = plsc.get_sparse_core_info()
    NC = info.num_cores
    rows_per_w = idx3.shape[1]
    mesh = plsc.VectorSubcoreMesh(core_axis_name="c", subcore_axis_name="s")

    @functools.partial(
        pl.kernel,
        mesh=mesh,
        out_type=(),
        scratch_types=[
            pltpu.VMEM((rows_per_w, idx3.shape[2]), jnp.int32),
            pltpu.VMEM_SHARED((VOCAB, LANE), jnp.float32),
            [pltpu.VMEM((idx3.shape[2], LANE), jnp.float32)] * NBUF,
            [pltpu.SemaphoreType.DMA] * NBUF,
            [pltpu.SemaphoreType.DMA] * NBUF,
        ],
    )
    def k(idx_hbm, table_hbm, out_hbm, idx_v, tbl_sh, rows, gsem, wsem):
        sid = lax.axis_index("s")
        wid = sid * NC + lax.axis_index("c")
        base = wid * rows_per_w

        @pl.when(sid == 0)
        def _():
            pltpu.sync_copy(table_hbm, tbl_sh)

        pltpu.sync_copy(idx_hbm.at[wid], idx_v)
        plsc.subcore_barrier()

        pltpu.async_copy(tbl_sh.at[idx_v.at[0]], rows[0], gsem[0])
        pltpu.async_copy(tbl_sh.at[idx_v.at[1]], rows[1], gsem[1])

        def body(i, carry):
            for s in range(NBUF):
                j = NBUF * i + s
                cur = rows[s]
                nxt = rows[(s + 2) % NBUF]

                @pl.when(j + 2 < rows_per_w)
                def _():
                    @pl.when(j >= 2)
                    def _():
                        pltpu.make_async_copy(
                            nxt,
                            out_hbm.at[0].at[:, pl.ds(0, LANE)],
                            wsem[(s + 2) % NBUF],
                        ).wait()

                    pltpu.async_copy(
                        tbl_sh.at[idx_v.at[j + 2]], nxt, gsem[(s + 2) % NBUF]
                    )

                pltpu.make_async_copy(
                    tbl_sh.at[idx_v.at[j]], cur, gsem[s]
                ).wait()
                pltpu.async_copy(
                    cur, out_hbm.at[base + j].at[:, pl.ds(0, LANE)], wsem[s]
                )
            return carry

        lax.fori_loop(0, rows_per_w // NBUF, body, 0, unroll=False)
        for s in range(NBUF):
            pltpu.make_async_copy(
                rows[s], out_hbm.at[0].at[:, pl.ds(0, LANE)], wsem[s]
            ).wait()

    k(idx3, table_a, out_ref)


def _tc_tail(x, tb_hi, tb_lo, n_rows, chunk):
    grid = n_rows // TC_ROWS

    def body(x_ref, tb_hi_ref, tb_lo_ref, out_ref, vals_ref, sem):
        i = pl.program_id(0)
        idx = x_ref[...]
        onehot = (
            idx[:, :, None]
            == lax.broadcasted_iota(jnp.int32, (TC_ROWS, chunk, LANE), 2)
        ).astype(jnp.bfloat16)
        dn = (((2,), (0,)), ((), ()))
        vals_ref[...] = lax.dot_general(
            onehot, tb_hi_ref[...], dn, preferred_element_type=jnp.float32
        ) + lax.dot_general(
            onehot, tb_lo_ref[...], dn, preferred_element_type=jnp.float32
        )
        copy = pltpu.make_async_copy(
            vals_ref,
            out_ref.at[pl.ds(i * TC_ROWS, TC_ROWS), :, pl.ds(LANE, TAIL)],
            sem,
        )
        copy.start()
        copy.wait()

    return pl.pallas_call(
        body,
        grid=(grid,),
        in_specs=[
            pl.BlockSpec((TC_ROWS, chunk), lambda i: (i, 0)),
            pl.BlockSpec((LANE, TAIL), lambda i: (0, 0)),
            pl.BlockSpec((LANE, TAIL), lambda i: (0, 0)),
        ],
        out_specs=pl.BlockSpec(memory_space=pltpu.HBM),
        out_shape=jax.ShapeDtypeStruct((n_rows, chunk, EMBED_DIM), jnp.float32),
        scratch_shapes=[
            pltpu.VMEM((TC_ROWS, chunk, TAIL), jnp.float32),
            pltpu.SemaphoreType.DMA,
        ],
    )(x, tb_hi, tb_lo)


def kernel(x, atom_fea):
    n_rows, chunk = x.shape
    info = plsc.get_sparse_core_info()
    NW = info.num_cores * info.num_subcores
    xi = x.astype(jnp.int32)
    idx3 = xi.reshape(NW, n_rows // NW, chunk)
    table_a = atom_fea[:, :LANE]
    tail_f32 = jnp.zeros((LANE, TAIL), jnp.float32)
    tail_f32 = lax.dynamic_update_slice(
        tail_f32, atom_fea[:, LANE:EMBED_DIM], (0, 0)
    )
    tb_hi = tail_f32.astype(jnp.bfloat16)
    tb_lo = (tail_f32 - tb_hi.astype(jnp.float32)).astype(jnp.bfloat16)
    out = _tc_tail(xi, tb_hi, tb_lo, n_rows, chunk)
    out_ref = jax.new_ref(out)
    _sc_head_inplace(out_ref, idx3, table_a)
    return jax.freeze(out_ref)


# speedup vs baseline: 1.4117x; 1.4117x over previous
import functools

import jax
import jax.numpy as jnp
from jax import lax
from jax.experimental import pallas as pl
from jax.experimental.pallas import tpu as pltpu
from jax.experimental.pallas import tpu_sc as plsc

VOCAB = 120
EMBED_DIM = 200
LANE = 128
WIDE = 2 * LANE
NBUF = 4


def _sc_gather_full(idx3, table_w, n_rows, chunk):
    info = plsc.get_sparse_core_info()
    NC = info.num_cores
    rows_per_w = idx3.shape[1]
    mesh = plsc.VectorSubcoreMesh(core_axis_name="c", subcore_axis_name="s")

    @functools.partial(
        pl.kernel,
        mesh=mesh,
        out_type=jax.ShapeDtypeStruct((n_rows, chunk, WIDE), jnp.float32),
        scratch_types=[
            pltpu.VMEM((rows_per_w, idx3.shape[2]), jnp.int32),
            pltpu.VMEM_SHARED((VOCAB, LANE), jnp.float32),
            pltpu.VMEM_SHARED((VOCAB, LANE), jnp.float32),
            [pltpu.VMEM((idx3.shape[2], LANE), jnp.float32)] * NBUF,
            [pltpu.VMEM((idx3.shape[2], LANE), jnp.float32)] * NBUF,
            [pltpu.SemaphoreType.DMA] * NBUF,
            [pltpu.SemaphoreType.DMA] * NBUF,
            [pltpu.SemaphoreType.DMA] * NBUF,
            [pltpu.SemaphoreType.DMA] * NBUF,
        ],
    )
    def k(
        idx_hbm,
        table_hbm,
        out_hbm,
        idx_v,
        tbl_a,
        tbl_b,
        rows_a,
        rows_b,
        gsem_a,
        gsem_b,
        wsem_a,
        wsem_b,
    ):
        sid = lax.axis_index("s")
        wid = sid * NC + lax.axis_index("c")
        base = wid * rows_per_w

        @pl.when(sid == 0)
        def _():
            pltpu.sync_copy(table_hbm.at[:, pl.ds(0, LANE)], tbl_a)

        @pl.when(sid == 1)
        def _():
            pltpu.sync_copy(table_hbm.at[:, pl.ds(LANE, LANE)], tbl_b)

        pltpu.sync_copy(idx_hbm.at[wid], idx_v)
        plsc.subcore_barrier()

        for p in range(2):
            pltpu.async_copy(tbl_a.at[idx_v.at[p]], rows_a[p], gsem_a[p])
            pltpu.async_copy(tbl_b.at[idx_v.at[p]], rows_b[p], gsem_b[p])

        def body(i, carry):
            for s in range(NBUF):
                j = NBUF * i + s
                n = (s + 2) % NBUF

                @pl.when(j + 2 < rows_per_w)
                def _():
                    @pl.when(j >= 2)
                    def _():
                        pltpu.make_async_copy(
                            rows_a[n],
                            out_hbm.at[0].at[:, pl.ds(0, LANE)],
                            wsem_a[n],
                        ).wait()
                        pltpu.make_async_copy(
                            rows_b[n],
                            out_hbm.at[0].at[:, pl.ds(LANE, LANE)],
                            wsem_b[n],
                        ).wait()

                    pltpu.async_copy(
                        tbl_a.at[idx_v.at[j + 2]], rows_a[n], gsem_a[n]
                    )
                    pltpu.async_copy(
                        tbl_b.at[idx_v.at[j + 2]], rows_b[n], gsem_b[n]
                    )

                pltpu.make_async_copy(
                    tbl_a.at[idx_v.at[j]], rows_a[s], gsem_a[s]
                ).wait()
                pltpu.async_copy(
                    rows_a[s],
                    out_hbm.at[base + j].at[:, pl.ds(0, LANE)],
                    wsem_a[s],
                )
                pltpu.make_async_copy(
                    tbl_b.at[idx_v.at[j]], rows_b[s], gsem_b[s]
                ).wait()
                pltpu.async_copy(
                    rows_b[s],
                    out_hbm.at[base + j].at[:, pl.ds(LANE, LANE)],
                    wsem_b[s],
                )
            return carry

        lax.fori_loop(0, rows_per_w // NBUF, body, 0, unroll=False)
        for s in range(NBUF):
            pltpu.make_async_copy(
                rows_a[s], out_hbm.at[0].at[:, pl.ds(0, LANE)], wsem_a[s]
            ).wait()
            pltpu.make_async_copy(
                rows_b[s], out_hbm.at[0].at[:, pl.ds(LANE, LANE)], wsem_b[s]
            ).wait()

    return k(idx3, table_w)


def kernel(x, atom_fea):
    n_rows, chunk = x.shape
    info = plsc.get_sparse_core_info()
    NW = info.num_cores * info.num_subcores
    xi = x.astype(jnp.int32)
    idx3 = xi.reshape(NW, n_rows // NW, chunk)
    table_w = jnp.zeros((VOCAB, WIDE), jnp.float32)
    table_w = lax.dynamic_update_slice(table_w, atom_fea, (0, 0))
    out_w = _sc_gather_full(idx3, table_w, n_rows, chunk)
    return out_w[:, :, :EMBED_DIM]
